# baseline (speedup 1.0000x reference)
import functools
import math

import jax
import jax.numpy as jnp
from jax.experimental import pallas as pl
from jax.experimental.pallas import tpu as pltpu

_NH = 12
_WT = 128


def _fused_kernel(x_ref, wq_ref, wk_ref, wv_ref, wo_ref, bq_ref, bk_ref,
                  bv_ref, bo_ref, g_ref, be_ref, o_ref, *, nb, sb, scale,
                  eps):
    x = x_ref[...]
    xb = x.astype(jnp.bfloat16)
    dn_in = (((1,), (0,)), ((), ()))

    q = jax.lax.dot_general(xb, wq_ref[...], dn_in,
                            preferred_element_type=jnp.float32) + bq_ref[...]
    k = jax.lax.dot_general(xb, wk_ref[...], dn_in,
                            preferred_element_type=jnp.float32) + bk_ref[...]
    v = jax.lax.dot_general(xb, wv_ref[...], dn_in,
                            preferred_element_type=jnp.float32) + bv_ref[...]

    qb = (q * scale).astype(jnp.bfloat16)
    kb = k.astype(jnp.bfloat16)
    vb = v.astype(jnp.bfloat16)

    lane = jax.lax.broadcasted_iota(jnp.int32, (sb, _WT), 1)
    lo_half = lane < (_WT // 2)

    row_blocks = []
    for b in range(nb):
        rows = slice(b * sb, (b + 1) * sb)
        head_parts = []
        for h in range(_NH):
            tile = slice(h * _WT, (h + 1) * _WT)
            qh = qb[rows, tile]
            kh = kb[rows, tile]
            s = jax.lax.dot_general(qh, kh, (((1,), (1,)), ((), ())),
                                    preferred_element_type=jnp.float32)
            s = s - jnp.max(s, axis=-1, keepdims=True)
            pb = jnp.exp(s).astype(jnp.bfloat16)
            nd = jnp.dot(pb, vb[rows, tile],
                         preferred_element_type=jnp.float32)
            den = jnp.where(lo_half, jnp.roll(nd, _WT // 2, axis=1), nd)
            head_parts.append((nd / den).astype(jnp.bfloat16))
        row_blocks.append(jnp.concatenate(head_parts, axis=1))
    ctxb = jnp.concatenate(row_blocks, axis=0)

    h_out = jax.lax.dot_general(ctxb, wo_ref[...], dn_in,
                                preferred_element_type=jnp.float32)
    h_out = h_out + bo_ref[...] + x
    mean = jnp.mean(h_out, axis=-1, keepdims=True)
    c = h_out - mean
    var = jnp.mean(c * c, axis=-1, keepdims=True)
    y = c * jax.lax.rsqrt(var + eps) * g_ref[...] + be_ref[...]
    o_ref[...] = y.astype(o_ref.dtype)


def kernel(hidden_states, wq, wk, wv, wo, bq, bk, bv, bo, gamma, beta):
    B, S, H = hidden_states.shape
    nh = _NH
    dh = H // nh
    M = B * S
    W = nh * _WT
    pad = _WT - dh
    dtype = hidden_states.dtype

    nb = 4
    while B % nb:
        nb -= 1
    tm = nb * S
    grid = (B // nb,)

    x2 = hidden_states.reshape(M, H)

    def widen_in(w):
        wt = w.T.reshape(H, nh, dh)
        wt = jnp.pad(wt, ((0, 0), (0, 0), (0, pad)))
        return wt.reshape(H, W).astype(jnp.bfloat16)

    wqw = widen_in(wq)
    wkw = widen_in(wk)
    wvw = widen_in(wv)
    wot = wo.T.reshape(nh, dh, H)
    wot = jnp.pad(wot, ((0, 0), (0, pad), (0, 0)))
    wow = wot.reshape(W, H).astype(jnp.bfloat16)

    def widen_b(b_vec, fill):
        bt = b_vec.reshape(nh, dh).astype(jnp.float32)
        bt = jnp.pad(bt, ((0, 0), (0, pad)), constant_values=fill)
        return bt.reshape(1, W)

    bqw = widen_b(bq, 0.0)
    bkw = widen_b(bk, 0.0)
    bvw = widen_b(bv, 1.0)
    bo2 = bo.reshape(1, H).astype(jnp.float32)
    g2 = gamma.reshape(1, H).astype(jnp.float32)
    be2 = beta.reshape(1, H).astype(jnp.float32)

    row_spec = pl.BlockSpec((tm, H), lambda i: (i, 0))
    win_spec = pl.BlockSpec((H, W), lambda i: (0, 0))
    wout_spec = pl.BlockSpec((W, H), lambda i: (0, 0))
    vecw_spec = pl.BlockSpec((1, W), lambda i: (0, 0))
    vec_spec = pl.BlockSpec((1, H), lambda i: (0, 0))

    out = pl.pallas_call(
        functools.partial(_fused_kernel, nb=nb, sb=S,
                          scale=1.0 / math.sqrt(dh), eps=1e-12),
        out_shape=jax.ShapeDtypeStruct((M, H), dtype),
        grid=grid,
        in_specs=[row_spec, win_spec, win_spec, win_spec, wout_spec,
                  vecw_spec, vecw_spec, vecw_spec, vec_spec, vec_spec,
                  vec_spec],
        out_specs=row_spec,
        compiler_params=pltpu.CompilerParams(
            dimension_semantics=("parallel",),
            vmem_limit_bytes=56 * 1024 * 1024,
        ),
    )(x2, wqw, wkw, wvw, wow, bqw, bkw, bvw, bo2, g2, be2)

    return out.reshape(B, S, H)


# speedup vs baseline: 1.3175x; 1.3175x over previous
import functools
import math

import jax
import jax.numpy as jnp
from jax.experimental import pallas as pl
from jax.experimental.pallas import tpu as pltpu

_NH = 12


def _fused_kernel(x_ref, wq_ref, wk_ref, wv_ref, wo_ref, bq_ref, bk_ref,
                  bv_ref, bo_ref, g_ref, be_ref, o_ref, *, nb, sb, dh, scale,
                  eps):
    x = x_ref[...]
    xb = x.astype(jnp.bfloat16)
    dn = (((1,), (1,)), ((), ()))

    q = jax.lax.dot_general(xb, wq_ref[...], dn,
                            preferred_element_type=jnp.float32) + bq_ref[...]
    k = jax.lax.dot_general(xb, wk_ref[...], dn,
                            preferred_element_type=jnp.float32) + bk_ref[...]
    v = jax.lax.dot_general(xb, wv_ref[...], dn,
                            preferred_element_type=jnp.float32) + bv_ref[...]

    qb = (q * scale).astype(jnp.bfloat16)
    kb = k.astype(jnp.bfloat16)
    vb = v.astype(jnp.bfloat16)

    ones_dh = jnp.ones((sb, dh), dtype=jnp.bfloat16)
    row_blocks = []
    for b in range(nb):
        rows = slice(b * sb, (b + 1) * sb)
        head_parts = []
        for h in range(_NH):
            cols = slice(h * dh, (h + 1) * dh)
            qh = qb[rows, cols]
            kh = kb[rows, cols]
            s = jax.lax.dot_general(qh, kh, dn,
                                    preferred_element_type=jnp.float32)
            s = s - jnp.max(s, axis=-1, keepdims=True)
            pb = jnp.exp(s).astype(jnp.bfloat16)
            num = jnp.dot(pb, vb[rows, cols],
                          preferred_element_type=jnp.float32)
            den = jnp.dot(pb, ones_dh,
                          preferred_element_type=jnp.float32)
            head_parts.append((num / den).astype(jnp.bfloat16))
        row_blocks.append(jnp.concatenate(head_parts, axis=1))
    ctxb = jnp.concatenate(row_blocks, axis=0)

    h_out = jax.lax.dot_general(ctxb, wo_ref[...], dn,
                                preferred_element_type=jnp.float32)
    h_out = h_out + bo_ref[...] + x
    mean = jnp.mean(h_out, axis=-1, keepdims=True)
    c = h_out - mean
    var = jnp.mean(c * c, axis=-1, keepdims=True)
    y = c * jax.lax.rsqrt(var + eps) * g_ref[...] + be_ref[...]
    o_ref[...] = y.astype(o_ref.dtype)


def kernel(hidden_states, wq, wk, wv, wo, bq, bk, bv, bo, gamma, beta):
    B, S, H = hidden_states.shape
    nh = _NH
    dh = H // nh
    M = B * S
    dtype = hidden_states.dtype

    nb = 1
    while B % nb:
        nb -= 1
    tm = nb * S
    grid = (B // nb,)

    x2 = hidden_states.reshape(M, H)
    wqb = wq.astype(jnp.bfloat16)
    wkb = wk.astype(jnp.bfloat16)
    wvb = wv.astype(jnp.bfloat16)
    wob = wo.astype(jnp.bfloat16)
    bq2 = bq.reshape(1, H).astype(jnp.float32)
    bk2 = bk.reshape(1, H).astype(jnp.float32)
    bv2 = bv.reshape(1, H).astype(jnp.float32)
    bo2 = bo.reshape(1, H).astype(jnp.float32)
    g2 = gamma.reshape(1, H).astype(jnp.float32)
    be2 = beta.reshape(1, H).astype(jnp.float32)

    row_spec = pl.BlockSpec((tm, H), lambda i: (i, 0))
    wt_spec = pl.BlockSpec((H, H), lambda i: (0, 0))
    vec_spec = pl.BlockSpec((1, H), lambda i: (0, 0))

    out = pl.pallas_call(
        functools.partial(_fused_kernel, nb=nb, sb=S, dh=dh,
                          scale=1.0 / math.sqrt(dh), eps=1e-12),
        out_shape=jax.ShapeDtypeStruct((M, H), dtype),
        grid=grid,
        in_specs=[row_spec, wt_spec, wt_spec, wt_spec, wt_spec,
                  vec_spec, vec_spec, vec_spec, vec_spec, vec_spec, vec_spec],
        out_specs=row_spec,
        compiler_params=pltpu.CompilerParams(
            dimension_semantics=("parallel",),
            vmem_limit_bytes=48 * 1024 * 1024,
        ),
    )(x2, wqb, wkb, wvb, wob, bq2, bk2, bv2, bo2, g2, be2)

    return out.reshape(B, S, H)


# speedup vs baseline: 2.4345x; 1.8478x over previous
import functools
import math

import jax
import jax.numpy as jnp
from jax.experimental import pallas as pl
from jax.experimental.pallas import tpu as pltpu

_NH = 12


def _fused_kernel(x_ref, wq_ref, wk_ref, wv_ref, wo_ref, bq_ref, bk_ref,
                  bv_ref, bo_ref, g_ref, be_ref, o_ref, *, nb, sb, dh, scale,
                  eps):
    x = x_ref[...]
    xb = x.astype(jnp.bfloat16)
    dn = (((1,), (1,)), ((), ()))

    q = jax.lax.dot_general(xb, wq_ref[...], dn,
                            preferred_element_type=jnp.float32) + bq_ref[...]
    k = jax.lax.dot_general(xb, wk_ref[...], dn,
                            preferred_element_type=jnp.float32) + bk_ref[...]
    v = jax.lax.dot_general(xb, wv_ref[...], dn,
                            preferred_element_type=jnp.float32) + bv_ref[...]

    qb = (q * scale).astype(jnp.bfloat16)
    kb = k.astype(jnp.bfloat16)
    vb = v.astype(jnp.bfloat16)

    ones_dh = jnp.ones((sb, dh), dtype=jnp.bfloat16)
    row_blocks = []
    for b in range(nb):
        rows = slice(b * sb, (b + 1) * sb)
        head_parts = []
        for h in range(_NH):
            cols = slice(h * dh, (h + 1) * dh)
            qh = qb[rows, cols]
            kh = kb[rows, cols]
            s = jax.lax.dot_general(qh, kh, dn,
                                    preferred_element_type=jnp.float32)
            s = s - jnp.max(s, axis=-1, keepdims=True)
            pb = jnp.exp(s).astype(jnp.bfloat16)
            num = jnp.dot(pb, vb[rows, cols],
                          preferred_element_type=jnp.float32)
            den = jnp.dot(pb, ones_dh,
                          preferred_element_type=jnp.float32)
            head_parts.append((num / den).astype(jnp.bfloat16))
        row_blocks.append(jnp.concatenate(head_parts, axis=1))
    ctxb = jnp.concatenate(row_blocks, axis=0)

    h_out = jax.lax.dot_general(ctxb, wo_ref[...], dn,
                                preferred_element_type=jnp.float32)
    h_out = h_out + bo_ref[...] + x
    mean = jnp.mean(h_out, axis=-1, keepdims=True)
    c = h_out - mean
    var = jnp.mean(c * c, axis=-1, keepdims=True)
    y = c * jax.lax.rsqrt(var + eps) * g_ref[...] + be_ref[...]
    o_ref[...] = y.astype(o_ref.dtype)


def kernel(hidden_states, wq, wk, wv, wo, bq, bk, bv, bo, gamma, beta):
    B, S, H = hidden_states.shape
    nh = _NH
    dh = H // nh
    M = B * S
    dtype = hidden_states.dtype

    nb = 8
    while B % nb:
        nb -= 1
    tm = nb * S
    grid = (B // nb,)

    x2 = hidden_states.reshape(M, H)
    wqb = wq.astype(jnp.bfloat16)
    wkb = wk.astype(jnp.bfloat16)
    wvb = wv.astype(jnp.bfloat16)
    wob = wo.astype(jnp.bfloat16)
    bq2 = bq.reshape(1, H).astype(jnp.float32)
    bk2 = bk.reshape(1, H).astype(jnp.float32)
    bv2 = bv.reshape(1, H).astype(jnp.float32)
    bo2 = bo.reshape(1, H).astype(jnp.float32)
    g2 = gamma.reshape(1, H).astype(jnp.float32)
    be2 = beta.reshape(1, H).astype(jnp.float32)

    row_spec = pl.BlockSpec((tm, H), lambda i: (i, 0))
    wt_spec = pl.BlockSpec((H, H), lambda i: (0, 0))
    vec_spec = pl.BlockSpec((1, H), lambda i: (0, 0))

    out = pl.pallas_call(
        functools.partial(_fused_kernel, nb=nb, sb=S, dh=dh,
                          scale=1.0 / math.sqrt(dh), eps=1e-12),
        out_shape=jax.ShapeDtypeStruct((M, H), dtype),
        grid=grid,
        in_specs=[row_spec, wt_spec, wt_spec, wt_spec, wt_spec,
                  vec_spec, vec_spec, vec_spec, vec_spec, vec_spec, vec_spec],
        out_specs=row_spec,
        compiler_params=pltpu.CompilerParams(
            dimension_semantics=("parallel",),
            vmem_limit_bytes=48 * 1024 * 1024,
        ),
    )(x2, wqb, wkb, wvb, wob, bq2, bk2, bv2, bo2, g2, be2)

    return out.reshape(B, S, H)
